# baseline (speedup 1.0000x reference)
import functools

import jax
import jax.numpy as jnp
from jax import lax
from jax.experimental import pallas as pl
from jax.experimental.pallas import tpu as pltpu
from jax.experimental.pallas import tpu_sc as plsc

B = 16384
D = 32
L = 16
NC = 2
NS = 16
NW = NC * NS
BPW = B // NW
W = 128
NCHUNK = BPW // W


@functools.partial(
    pl.kernel,
    out_type=jax.ShapeDtypeStruct((B,), jnp.float32),
    mesh=plsc.VectorSubcoreMesh(core_axis_name="c", subcore_axis_name="s"),
    compiler_params=pltpu.CompilerParams(
        needs_layout_passes=False, use_tc_tiling_on_sc=False),
    scratch_types=[
        pltpu.VMEM((BPW,), jnp.int32),
        pltpu.VMEM((BPW,), jnp.int32),
        pltpu.VMEM((D, BPW), jnp.float32),
        pltpu.VMEM((D, BPW), jnp.float32),
        pltpu.VMEM((BPW,), jnp.float32),
        pltpu.SemaphoreType.DMA,
        pltpu.SemaphoreType.DMA,
    ],
)
def _cf_sc(user_hbm, item_hbm, uembT_hbm, iembT_hbm, out_hbm,
           uidx, iidx, ucols, icols, outv, sem_u, sem_i):
    wid = lax.axis_index("s") * NC + lax.axis_index("c")
    base = wid * BPW
    pltpu.sync_copy(user_hbm.at[pl.ds(base, BPW)], uidx)
    pltpu.sync_copy(item_hbm.at[pl.ds(base, BPW)], iidx)

    copies = []
    for d in range(D):
        for j in range(NCHUNK):
            copies.append(pltpu.async_copy(
                uembT_hbm.at[d].at[uidx.at[pl.ds(j * W, W)]],
                ucols.at[d].at[pl.ds(j * W, W)], sem_u))
            copies.append(pltpu.async_copy(
                iembT_hbm.at[d].at[iidx.at[pl.ds(j * W, W)]],
                icols.at[d].at[pl.ds(j * W, W)], sem_i))
    pltpu.make_async_copy(user_hbm, ucols, sem_u).wait()
    pltpu.make_async_copy(item_hbm, icols, sem_i).wait()

    def group(g, carry):
        gbase = pl.multiple_of(g * L, L)
        acc = jnp.zeros((L,), jnp.float32)
        for d in range(D):
            u = ucols[d, pl.ds(gbase, L)]
            v = icols[d, pl.ds(gbase, L)]
            acc = acc + u * v
        outv[pl.ds(gbase, L)] = acc
        return carry

    lax.fori_loop(0, BPW // L, group, 0)
    pltpu.sync_copy(outv, out_hbm.at[pl.ds(base, BPW)])


def kernel(user, item, user_emb, item_emb):
    return _cf_sc(user, item, user_emb.T, item_emb.T)


# speedup vs baseline: 5.6108x; 5.6108x over previous
import functools

import jax
import jax.numpy as jnp
from jax import lax
from jax.experimental import pallas as pl
from jax.experimental.pallas import tpu as pltpu
from jax.experimental.pallas import tpu_sc as plsc

B = 16384
D = 32
L = 16
NC = 2
NS = 16
NW = NC * NS
BPW = B // NW
RPB = 128 // D
NQ = 1000000 // RPB
HALF = BPW // 2
W = 128


@functools.partial(
    pl.kernel,
    out_type=jax.ShapeDtypeStruct((B,), jnp.float32),
    mesh=plsc.VectorSubcoreMesh(core_axis_name="c", subcore_axis_name="s"),
    compiler_params=pltpu.CompilerParams(
        needs_layout_passes=False, use_tc_tiling_on_sc=True),
    scratch_types=[
        pltpu.VMEM((BPW,), jnp.int32),
        pltpu.VMEM((BPW,), jnp.int32),
        pltpu.VMEM((BPW,), jnp.int32),
        pltpu.VMEM((BPW,), jnp.int32),
        pltpu.VMEM((HALF, 128), jnp.float32),
        pltpu.VMEM((HALF, 128), jnp.float32),
        pltpu.VMEM((BPW,), jnp.float32),
        pltpu.SemaphoreType.DMA,
    ],
)
def _cf_sc(user_hbm, item_hbm, uemb_hbm, iemb_hbm, out_hbm,
           uidx, iidx, uq, iq, ubig, ibig, outv, sem):
    wid = lax.axis_index("s") * NC + lax.axis_index("c")
    base = wid * BPW
    pltpu.sync_copy(user_hbm.at[pl.ds(base, BPW)], uidx)
    pltpu.sync_copy(item_hbm.at[pl.ds(base, BPW)], iidx)

    def qbuild(c, carry):
        cbase = pl.multiple_of(c * L, L)
        uq[pl.ds(cbase, L)] = lax.shift_right_logical(uidx[pl.ds(cbase, L)], 2)
        iq[pl.ds(cbase, L)] = lax.shift_right_logical(iidx[pl.ds(cbase, L)], 2)
        return carry

    lax.fori_loop(0, BPW // L, qbuild, 0)

    lanes = lax.iota(jnp.int32, L)
    for h in range(2):
        hbase = h * HALF
        copies = []
        for j in range(HALF // W):
            copies.append(pltpu.async_copy(
                uemb_hbm.at[uq.at[pl.ds(hbase + j * W, W)]],
                ubig.at[pl.ds(j * W, W)], sem))
            copies.append(pltpu.async_copy(
                iemb_hbm.at[iq.at[pl.ds(hbase + j * W, W)]],
                ibig.at[pl.ds(j * W, W)], sem))
        for cp in copies:
            cp.wait()

        def group(g, carry):
            gbase = pl.multiple_of(g * L, L)
            row = gbase + lanes
            usub = (uidx[pl.ds(hbase + gbase, L)] & (RPB - 1)) * D
            isub = (iidx[pl.ds(hbase + gbase, L)] & (RPB - 1)) * D
            acc = jnp.zeros((L,), jnp.float32)
            for d in range(D):
                u = plsc.load_gather(ubig, [row, usub + d])
                v = plsc.load_gather(ibig, [row, isub + d])
                acc = acc + u * v
            outv[pl.ds(hbase + gbase, L)] = acc
            return carry

        lax.fori_loop(0, HALF // L, group, 0)

    pltpu.sync_copy(outv, out_hbm.at[pl.ds(base, BPW)])


def kernel(user, item, user_emb, item_emb):
    ue2 = user_emb.reshape(NQ, 128)
    ie2 = item_emb.reshape(NQ, 128)
    return _cf_sc(user, item, ue2, ie2)


# speedup vs baseline: 19.3585x; 3.4502x over previous
import functools

import jax
import jax.numpy as jnp
from jax import lax
from jax.experimental import pallas as pl
from jax.experimental.pallas import tpu as pltpu
from jax.experimental.pallas import tpu_sc as plsc

B = 16384
D = 32
L = 16
NC = 2
NS = 16
NW = NC * NS
BPW = B // NW
NV = 1000000
FPT = D // NS
MAIN = (NV // 128) * 128
TAIL = NV - MAIN
TB = D * MAIN
TW = D * NV
CH = 55552
NCH = MAIN // CH
W = 128
NWORDS = BPW * D
NDMA = NWORDS // W


@functools.partial(
    pl.kernel,
    out_type=(jax.ShapeDtypeStruct((TW,), jnp.float32),
              jax.ShapeDtypeStruct((TW,), jnp.float32)),
    mesh=plsc.VectorSubcoreMesh(core_axis_name="c", subcore_axis_name="s"),
    compiler_params=pltpu.CompilerParams(
        needs_layout_passes=False, use_tc_tiling_on_sc=True),
    scratch_types=[
        pltpu.VMEM((CH,), jnp.float32),
        pltpu.VMEM((CH,), jnp.float32),
        pltpu.VMEM((D, TAIL), jnp.float32),
        pltpu.VMEM((D * TAIL,), jnp.float32),
        pltpu.SemaphoreType.DMA,
        pltpu.SemaphoreType.DMA,
    ],
)
def _detile(uembT_hbm, iembT_hbm, uscr_hbm, iscr_hbm,
            buf0, buf1, tailbuf, tailflat, sem_in, sem_out):
    core = lax.axis_index("c")
    sub = lax.axis_index("s")

    def copy_rows(src, dst):
        def drain_in():
            pltpu.make_async_copy(
                dst.at[pl.ds(0, CH)], buf0, sem_in).wait()

        def drain_out():
            pltpu.make_async_copy(
                buf0, dst.at[pl.ds(0, CH)], sem_out).wait()

        for d in range(D):

            @pl.when(sub == d // FPT)
            def _():
                def chunk(k, carry):
                    off = pl.multiple_of(k * CH, 128)
                    pltpu.sync_copy(src.at[d].at[pl.ds(off, CH)], buf0)
                    pltpu.sync_copy(buf0, dst.at[pl.ds(d * MAIN + off, CH)])
                    return carry

                lax.fori_loop(0, NCH, chunk, 0)

        @pl.when(sub == 0)
        def _():
            pltpu.sync_copy(src.at[:, pl.ds(MAIN, TAIL)], tailbuf)

            def flat(c, carry):
                cbase = pl.multiple_of(c * L, L)
                for d in range(D):
                    tailflat[pl.ds(d * TAIL + cbase, L)] = \
                        tailbuf[d, pl.ds(cbase, L)]
                return carry

            lax.fori_loop(0, TAIL // L, flat, 0)
            pltpu.sync_copy(tailflat, dst.at[pl.ds(TB, D * TAIL)])

    @pl.when(core == 0)
    def _():
        copy_rows(uembT_hbm, uscr_hbm)

    @pl.when(core == 1)
    def _():
        copy_rows(iembT_hbm, iscr_hbm)


@functools.partial(
    pl.kernel,
    out_type=jax.ShapeDtypeStruct((B,), jnp.float32),
    mesh=plsc.VectorSubcoreMesh(core_axis_name="c", subcore_axis_name="s"),
    compiler_params=pltpu.CompilerParams(
        needs_layout_passes=False, use_tc_tiling_on_sc=False),
    scratch_types=[
        pltpu.VMEM((BPW,), jnp.int32),
        pltpu.VMEM((BPW,), jnp.int32),
        pltpu.VMEM((NWORDS,), jnp.int32),
        pltpu.VMEM((NWORDS,), jnp.int32),
        pltpu.VMEM((NWORDS,), jnp.float32),
        pltpu.VMEM((NWORDS,), jnp.float32),
        pltpu.VMEM((BPW,), jnp.float32),
        pltpu.SemaphoreType.DMA,
        pltpu.SemaphoreType.DMA,
    ],
)
def _gather_dot(user_hbm, item_hbm, uflat_hbm, iflat_hbm, out_hbm,
                uidx, iidx, uwords, iwords, uvals, ivals, outv, sem_u, sem_i):
    wid = lax.axis_index("s") * NC + lax.axis_index("c")
    base = wid * BPW
    pltpu.sync_copy(user_hbm.at[pl.ds(base, BPW)], uidx)
    pltpu.sync_copy(item_hbm.at[pl.ds(base, BPW)], iidx)

    def build(c, carry):
        cbase = pl.multiple_of(c * L, L)
        u = uidx[pl.ds(cbase, L)]
        v = iidx[pl.ds(cbase, L)]
        ut = u < MAIN
        vt = v < MAIN
        ustep = jnp.where(ut, MAIN, TAIL)
        vstep = jnp.where(vt, MAIN, TAIL)
        uoff = jnp.where(ut, u, TB + u - MAIN)
        voff = jnp.where(vt, v, TB + v - MAIN)
        for d in range(D):
            uwords[pl.ds(d * BPW + cbase, L)] = uoff
            iwords[pl.ds(d * BPW + cbase, L)] = voff
            uoff = uoff + ustep
            voff = voff + vstep
        return carry

    lax.fori_loop(0, BPW // L, build, 0)

    def fire(k, carry):
        kbase = pl.multiple_of(k * W, W)
        pltpu.async_copy(uflat_hbm.at[uwords.at[pl.ds(kbase, W)]],
                         uvals.at[pl.ds(kbase, W)], sem_u)
        pltpu.async_copy(iflat_hbm.at[iwords.at[pl.ds(kbase, W)]],
                         ivals.at[pl.ds(kbase, W)], sem_i)
        return carry

    lax.fori_loop(0, NDMA, fire, 0)
    pltpu.make_async_copy(uflat_hbm.at[pl.ds(0, NWORDS)], uvals, sem_u).wait()
    pltpu.make_async_copy(iflat_hbm.at[pl.ds(0, NWORDS)], ivals, sem_i).wait()

    def group(g, carry):
        gbase = pl.multiple_of(g * L, L)
        acc = jnp.zeros((L,), jnp.float32)
        for d in range(D):
            u = uvals[pl.ds(d * BPW + gbase, L)]
            v = ivals[pl.ds(d * BPW + gbase, L)]
            acc = acc + u * v
        outv[pl.ds(gbase, L)] = acc
        return carry

    lax.fori_loop(0, BPW // L, group, 0)
    pltpu.sync_copy(outv, out_hbm.at[pl.ds(base, BPW)])


def kernel(user, item, user_emb, item_emb):
    uflat, iflat = _detile(user_emb.T, item_emb.T)
    return _gather_dot(user, item, uflat, iflat)


# speedup vs baseline: 20.3667x; 1.0521x over previous
import functools

import jax
import jax.numpy as jnp
from jax import lax
from jax.experimental import pallas as pl
from jax.experimental.pallas import tpu as pltpu
from jax.experimental.pallas import tpu_sc as plsc

B = 16384
D = 32
L = 16
NC = 2
NS = 16
NW = NC * NS
BPW = B // NW
NV = 1000000
FPT = D // NS
MAIN = (NV // 128) * 128
TAIL = NV - MAIN
TB = D * MAIN
TW = D * NV
CH = 55552
NCH = MAIN // CH
W = 128
NWORDS = BPW * D
NDMA = NWORDS // W


@functools.partial(
    pl.kernel,
    out_type=(jax.ShapeDtypeStruct((TW,), jnp.float32),
              jax.ShapeDtypeStruct((TW,), jnp.float32)),
    mesh=plsc.VectorSubcoreMesh(core_axis_name="c", subcore_axis_name="s"),
    compiler_params=pltpu.CompilerParams(
        needs_layout_passes=False, use_tc_tiling_on_sc=True),
    scratch_types=[
        pltpu.VMEM((CH,), jnp.float32),
        pltpu.VMEM((CH,), jnp.float32),
        pltpu.VMEM((D, TAIL), jnp.float32),
        pltpu.VMEM((D * TAIL,), jnp.float32),
        pltpu.SemaphoreType.DMA,
        pltpu.SemaphoreType.DMA,
        pltpu.SemaphoreType.DMA,
        pltpu.SemaphoreType.DMA,
    ],
)
def _detile(uembT_hbm, iembT_hbm, uscr_hbm, iscr_hbm,
            buf0, buf1, tailbuf, tailflat, si0, si1, so0, so1):
    core = lax.axis_index("c")
    sub = lax.axis_index("s")

    def copy_rows(src, dst):
        bufs = (buf0, buf1)
        sins = (si0, si1)
        souts = (so0, so1)

        def drain_in(p):
            pltpu.make_async_copy(
                dst.at[pl.ds(0, CH)], bufs[p], sins[p]).wait()

        def drain_out(p):
            pltpu.make_async_copy(
                bufs[p], dst.at[pl.ds(0, CH)], souts[p]).wait()

        for d in range(D):

            @pl.when(sub == d // FPT)
            def _():
                def chunk(k, carry):
                    off = pl.multiple_of(k * CH, 128)
                    prev = pl.multiple_of((k - 1) * CH, 128)
                    for p in range(2):
                        q = 1 - p

                        @pl.when(k % 2 == p)
                        def _():
                            @pl.when(k >= 2)
                            def _():
                                drain_out(p)

                            pltpu.async_copy(
                                src.at[d].at[pl.ds(off, CH)],
                                bufs[p], sins[p])

                            @pl.when(k >= 1)
                            def _():
                                drain_in(q)
                                pltpu.async_copy(
                                    bufs[q],
                                    dst.at[pl.ds(d * MAIN + prev, CH)],
                                    souts[q])

                    return carry

                lax.fori_loop(0, NCH, chunk, 0)
                lastp = (NCH - 1) % 2
                drain_in(lastp)
                pltpu.async_copy(
                    bufs[lastp],
                    dst.at[pl.ds(d * MAIN + (NCH - 1) * CH, CH)],
                    souts[lastp])
                drain_out(1 - lastp)
                drain_out(lastp)

        @pl.when(sub == 0)
        def _():
            pltpu.sync_copy(src.at[:, pl.ds(MAIN, TAIL)], tailbuf)

            def flat(c, carry):
                cbase = pl.multiple_of(c * L, L)
                for d in range(D):
                    tailflat[pl.ds(d * TAIL + cbase, L)] = \
                        tailbuf[d, pl.ds(cbase, L)]
                return carry

            lax.fori_loop(0, TAIL // L, flat, 0)
            pltpu.sync_copy(tailflat, dst.at[pl.ds(TB, D * TAIL)])

    @pl.when(core == 0)
    def _():
        copy_rows(uembT_hbm, uscr_hbm)

    @pl.when(core == 1)
    def _():
        copy_rows(iembT_hbm, iscr_hbm)


@functools.partial(
    pl.kernel,
    out_type=jax.ShapeDtypeStruct((B,), jnp.float32),
    mesh=plsc.VectorSubcoreMesh(core_axis_name="c", subcore_axis_name="s"),
    compiler_params=pltpu.CompilerParams(
        needs_layout_passes=False, use_tc_tiling_on_sc=False),
    scratch_types=[
        pltpu.VMEM((BPW,), jnp.int32),
        pltpu.VMEM((BPW,), jnp.int32),
        pltpu.VMEM((NWORDS,), jnp.int32),
        pltpu.VMEM((NWORDS,), jnp.int32),
        pltpu.VMEM((NWORDS,), jnp.float32),
        pltpu.VMEM((NWORDS,), jnp.float32),
        pltpu.VMEM((BPW,), jnp.float32),
        pltpu.SemaphoreType.DMA,
        pltpu.SemaphoreType.DMA,
    ],
)
def _gather_dot(user_hbm, item_hbm, uflat_hbm, iflat_hbm, out_hbm,
                uidx, iidx, uwords, iwords, uvals, ivals, outv, sem_u, sem_i):
    wid = lax.axis_index("s") * NC + lax.axis_index("c")
    base = wid * BPW
    pltpu.sync_copy(user_hbm.at[pl.ds(base, BPW)], uidx)
    pltpu.sync_copy(item_hbm.at[pl.ds(base, BPW)], iidx)

    def build(c, carry):
        cbase = pl.multiple_of(c * L, L)
        u = uidx[pl.ds(cbase, L)]
        v = iidx[pl.ds(cbase, L)]
        ut = u < MAIN
        vt = v < MAIN
        ustep = jnp.where(ut, MAIN, TAIL)
        vstep = jnp.where(vt, MAIN, TAIL)
        uoff = jnp.where(ut, u, TB + u - MAIN)
        voff = jnp.where(vt, v, TB + v - MAIN)
        for d in range(D):
            uwords[pl.ds(d * BPW + cbase, L)] = uoff
            iwords[pl.ds(d * BPW + cbase, L)] = voff
            uoff = uoff + ustep
            voff = voff + vstep
        return carry

    lax.fori_loop(0, BPW // L, build, 0)

    def fire(k, carry):
        kbase = pl.multiple_of(k * W, W)
        pltpu.async_copy(uflat_hbm.at[uwords.at[pl.ds(kbase, W)]],
                         uvals.at[pl.ds(kbase, W)], sem_u)
        pltpu.async_copy(iflat_hbm.at[iwords.at[pl.ds(kbase, W)]],
                         ivals.at[pl.ds(kbase, W)], sem_i)
        return carry

    lax.fori_loop(0, NDMA, fire, 0)
    pltpu.make_async_copy(uflat_hbm.at[pl.ds(0, NWORDS)], uvals, sem_u).wait()
    pltpu.make_async_copy(iflat_hbm.at[pl.ds(0, NWORDS)], ivals, sem_i).wait()

    def group(g, carry):
        gbase = pl.multiple_of(g * L, L)
        acc = jnp.zeros((L,), jnp.float32)
        for d in range(D):
            u = uvals[pl.ds(d * BPW + gbase, L)]
            v = ivals[pl.ds(d * BPW + gbase, L)]
            acc = acc + u * v
        outv[pl.ds(gbase, L)] = acc
        return carry

    lax.fori_loop(0, BPW // L, group, 0)
    pltpu.sync_copy(outv, out_hbm.at[pl.ds(base, BPW)])


def kernel(user, item, user_emb, item_emb):
    uflat, iflat = _detile(user_emb.T, item_emb.T)
    return _gather_dot(user, item, uflat, iflat)


# speedup vs baseline: 21.0879x; 1.0354x over previous
import functools

import jax
import jax.numpy as jnp
from jax import lax
from jax.experimental import pallas as pl
from jax.experimental.pallas import tpu as pltpu
from jax.experimental.pallas import tpu_sc as plsc

B = 16384
D = 32
L = 16
NC = 2
NS = 16
NW = NC * NS
BPW = B // NW
NV = 1000000
FPT = D // NS
MAIN = (NV // 128) * 128
TAIL = NV - MAIN
TB = D * MAIN
TW = D * NV
QW = MAIN // 4
CW = 3968
NCK = QW // CW
W = 128
NWORDS = BPW * D
NDMA = NWORDS // W


@functools.partial(
    pl.kernel,
    out_type=(jax.ShapeDtypeStruct((TW,), jnp.float32),
              jax.ShapeDtypeStruct((TW,), jnp.float32)),
    mesh=plsc.VectorSubcoreMesh(core_axis_name="c", subcore_axis_name="s"),
    compiler_params=pltpu.CompilerParams(
        needs_layout_passes=False, use_tc_tiling_on_sc=True),
    scratch_types=[
        pltpu.VMEM((8, CW), jnp.float32),
        pltpu.VMEM((8, CW), jnp.float32),
        pltpu.VMEM((D, TAIL), jnp.float32),
        pltpu.VMEM((D * TAIL,), jnp.float32),
        pltpu.SemaphoreType.DMA,
        pltpu.SemaphoreType.DMA,
        pltpu.SemaphoreType.DMA,
        pltpu.SemaphoreType.DMA,
    ],
)
def _detile(uembT_hbm, iembT_hbm, uscr_hbm, iscr_hbm,
            buf0, buf1, tailbuf, tailflat, si0, si1, so0, so1):
    core = lax.axis_index("c")
    sub = lax.axis_index("s")

    def copy_rows(src, dst):
        bufs = (buf0, buf1)
        sins = (si0, si1)
        souts = (so0, so1)

        def drain_in(p):
            pltpu.make_async_copy(
                src.at[pl.ds(0, 8), pl.ds(0, CW)], bufs[p], sins[p]).wait()

        def drain_out(p):
            pltpu.make_async_copy(
                src.at[pl.ds(0, 8), pl.ds(0, CW)], bufs[p], souts[p]).wait()

        tr = sub // 4
        qq = sub % 4
        row0 = pl.multiple_of(tr * 8, 8)

        def outs(p, k):
            poff = pl.multiple_of(qq * QW + k * CW, 128)
            for s in range(8):
                pltpu.async_copy(
                    bufs[p].at[s],
                    dst.at[pl.ds((row0 + s) * MAIN + poff, CW)],
                    souts[p])

        def chunk(k, carry):
            coff = pl.multiple_of(qq * QW + k * CW, 128)
            for p in range(2):
                o = 1 - p

                @pl.when(k % 2 == p)
                def _():
                    @pl.when(k >= 2)
                    def _():
                        drain_out(p)

                    pltpu.async_copy(
                        src.at[pl.ds(row0, 8), pl.ds(coff, CW)],
                        bufs[p], sins[p])

                    @pl.when(k >= 1)
                    def _():
                        drain_in(o)
                        outs(o, k - 1)

            return carry

        lax.fori_loop(0, NCK, chunk, 0)
        lastp = (NCK - 1) % 2
        drain_in(lastp)
        outs(lastp, NCK - 1)
        drain_out(1 - lastp)
        drain_out(lastp)

        @pl.when(sub == 0)
        def _():
            pltpu.sync_copy(src.at[:, pl.ds(MAIN, TAIL)], tailbuf)

            def flat(c, carry):
                cbase = pl.multiple_of(c * L, L)
                for d in range(D):
                    tailflat[pl.ds(d * TAIL + cbase, L)] = \
                        tailbuf[d, pl.ds(cbase, L)]
                return carry

            lax.fori_loop(0, TAIL // L, flat, 0)
            pltpu.sync_copy(tailflat, dst.at[pl.ds(TB, D * TAIL)])

    @pl.when(core == 0)
    def _():
        copy_rows(uembT_hbm, uscr_hbm)

    @pl.when(core == 1)
    def _():
        copy_rows(iembT_hbm, iscr_hbm)


@functools.partial(
    pl.kernel,
    out_type=jax.ShapeDtypeStruct((B,), jnp.float32),
    mesh=plsc.VectorSubcoreMesh(core_axis_name="c", subcore_axis_name="s"),
    compiler_params=pltpu.CompilerParams(
        needs_layout_passes=False, use_tc_tiling_on_sc=False),
    scratch_types=[
        pltpu.VMEM((BPW,), jnp.int32),
        pltpu.VMEM((BPW,), jnp.int32),
        pltpu.VMEM((NWORDS,), jnp.int32),
        pltpu.VMEM((NWORDS,), jnp.int32),
        pltpu.VMEM((NWORDS,), jnp.float32),
        pltpu.VMEM((NWORDS,), jnp.float32),
        pltpu.VMEM((BPW,), jnp.float32),
        pltpu.SemaphoreType.DMA,
        pltpu.SemaphoreType.DMA,
    ],
)
def _gather_dot(user_hbm, item_hbm, uflat_hbm, iflat_hbm, out_hbm,
                uidx, iidx, uwords, iwords, uvals, ivals, outv, sem_u, sem_i):
    wid = lax.axis_index("s") * NC + lax.axis_index("c")
    base = wid * BPW
    pltpu.sync_copy(user_hbm.at[pl.ds(base, BPW)], uidx)
    pltpu.sync_copy(item_hbm.at[pl.ds(base, BPW)], iidx)

    def build(c, carry):
        cbase = pl.multiple_of(c * L, L)
        u = uidx[pl.ds(cbase, L)]
        v = iidx[pl.ds(cbase, L)]
        ut = u < MAIN
        vt = v < MAIN
        ustep = jnp.where(ut, MAIN, TAIL)
        vstep = jnp.where(vt, MAIN, TAIL)
        uoff = jnp.where(ut, u, TB + u - MAIN)
        voff = jnp.where(vt, v, TB + v - MAIN)
        for d in range(D):
            uwords[pl.ds(d * BPW + cbase, L)] = uoff
            iwords[pl.ds(d * BPW + cbase, L)] = voff
            uoff = uoff + ustep
            voff = voff + vstep
        return carry

    lax.fori_loop(0, BPW // L, build, 0)

    def fire(k, carry):
        kbase = pl.multiple_of(k * W, W)
        pltpu.async_copy(uflat_hbm.at[uwords.at[pl.ds(kbase, W)]],
                         uvals.at[pl.ds(kbase, W)], sem_u)
        pltpu.async_copy(iflat_hbm.at[iwords.at[pl.ds(kbase, W)]],
                         ivals.at[pl.ds(kbase, W)], sem_i)
        return carry

    lax.fori_loop(0, NDMA, fire, 0)
    pltpu.make_async_copy(uflat_hbm.at[pl.ds(0, NWORDS)], uvals, sem_u).wait()
    pltpu.make_async_copy(iflat_hbm.at[pl.ds(0, NWORDS)], ivals, sem_i).wait()

    def group(g, carry):
        gbase = pl.multiple_of(g * L, L)
        acc = jnp.zeros((L,), jnp.float32)
        for d in range(D):
            u = uvals[pl.ds(d * BPW + gbase, L)]
            v = ivals[pl.ds(d * BPW + gbase, L)]
            acc = acc + u * v
        outv[pl.ds(gbase, L)] = acc
        return carry

    lax.fori_loop(0, BPW // L, group, 0)
    pltpu.sync_copy(outv, out_hbm.at[pl.ds(base, BPW)])


def kernel(user, item, user_emb, item_emb):
    uflat, iflat = _detile(user_emb.T, item_emb.T)
    return _gather_dot(user, item, uflat, iflat)
